# Initial kernel scaffold; baseline (speedup 1.0000x reference)
import functools

import jax
import jax.numpy as jnp
from jax import lax
from jax.experimental import pallas as pl
from jax.experimental.pallas import tpu as pltpu
from jax.experimental.pallas import tpu_sc as plsc

_N = 10000
_D = 128
_L = 16
_NC = 2
_NS = 16
_NW = _NC * _NS
_CHUNK = 80


@functools.lru_cache(maxsize=None)
def _make_agg(E: int, with_counts: bool):
    e_per_tile = E // _NS
    n_chunks = e_per_tile // _CHUNK
    rows_per_tile = _N // _NS
    mesh = plsc.VectorSubcoreMesh(core_axis_name="c", subcore_axis_name="s")

    out_type = [
        jax.ShapeDtypeStruct((_N, _D), jnp.float32),
        jax.ShapeDtypeStruct((_N, _D), jnp.float32),
    ]
    scratch = [
        pltpu.VMEM((_CHUNK,), jnp.int32),
        pltpu.VMEM((_CHUNK,), jnp.int32),
        pltpu.VMEM((_CHUNK, _D), jnp.float32),
        pltpu.VMEM_SHARED((_N, _D), jnp.float32),
        pltpu.SemaphoreType.DMA,
    ]
    if with_counts:
        out_type += [
            jax.ShapeDtypeStruct((_N, _L), jnp.float32),
            jax.ShapeDtypeStruct((_N, _L), jnp.float32),
        ]
        scratch += [
            pltpu.VMEM((_CHUNK, _L), jnp.float32),
            pltpu.VMEM_SHARED((_N, _L), jnp.float32),
        ]

    def body(x_hbm, ei_hbm, zrow_hbm, zcnt_hbm, *rest):
        if with_counts:
            (sumf, sumb, cntf, cntb,
             gidx, sidx, rows, acc_sh, sem, ones, cnt_sh) = rest
        else:
            sumf, sumb, gidx, sidx, rows, acc_sh, sem = rest
            ones = cnt_sh = cntf = cntb = None

        c = lax.axis_index("c")
        s = lax.axis_index("s")
        row0 = s * rows_per_tile
        rs = pl.ds(row0, rows_per_tile)

        pltpu.sync_copy(zrow_hbm, acc_sh.at[rs])
        if with_counts:
            pltpu.sync_copy(zcnt_hbm, cnt_sh.at[rs])
            for i in range(_CHUNK):
                ones[i, :] = jnp.ones((_L,), jnp.float32)
        plsc.subcore_barrier()

        def edge_step(j, carry):
            off = s * e_per_tile + j * _CHUNK
            pltpu.sync_copy(ei_hbm.at[c, pl.ds(off, _CHUNK)], gidx)
            pltpu.sync_copy(ei_hbm.at[1 - c, pl.ds(off, _CHUNK)], sidx)
            pltpu.async_copy(x_hbm.at[gidx], rows, sem).wait()
            pltpu.sync_copy(rows, acc_sh.at[sidx], add=True)
            if with_counts:
                pltpu.sync_copy(ones, cnt_sh.at[sidx], add=True)
            return carry

        lax.fori_loop(0, n_chunks, edge_step, 0)
        plsc.subcore_barrier()

        @pl.when(c == 0)
        def _():
            pltpu.sync_copy(acc_sh.at[rs], sumf.at[rs])
            if with_counts:
                pltpu.sync_copy(cnt_sh.at[rs], cntf.at[rs])

        @pl.when(c == 1)
        def _():
            pltpu.sync_copy(acc_sh.at[rs], sumb.at[rs])
            if with_counts:
                pltpu.sync_copy(cnt_sh.at[rs], cntb.at[rs])

    return pl.kernel(body, out_type=tuple(out_type), mesh=mesh,
                     scratch_types=tuple(scratch))


def _layer_tc(sumf, cntf, sumb, cntb, x, Wl, Wr, WlT, WrT, bl, blT):
    BR = 1000
    grid = (_N // BR,)
    dn = (((1,), (1,)), ((), ()))

    def body(sf, cf, sb, cb, xr, wl, wr, wlt, wrt, b1, b2, o):
        inv_f = 1.0 / jnp.maximum(cf[:, 0:1], 1.0)
        inv_b = 1.0 / jnp.maximum(cb[:, 0:1], 1.0)
        hf = lax.dot_general(sf[...] * inv_f, wl[...], dn,
                             preferred_element_type=jnp.float32)
        hb = lax.dot_general(sb[...] * inv_b, wlt[...], dn,
                             preferred_element_type=jnp.float32)
        hx = lax.dot_general(xr[...], wr[...] + wrt[...], dn,
                             preferred_element_type=jnp.float32)
        o[...] = hf + hb + hx + (b1[...] + b2[...])

    row_spec = pl.BlockSpec((BR, _D), lambda i: (i, 0))
    cnt_spec = pl.BlockSpec((BR, _L), lambda i: (i, 0))
    w_spec = pl.BlockSpec((_D, _D), lambda i: (0, 0))
    b_spec = pl.BlockSpec((1, _D), lambda i: (0, 0))
    return pl.pallas_call(
        body,
        grid=grid,
        in_specs=[row_spec, cnt_spec, row_spec, cnt_spec, row_spec,
                  w_spec, w_spec, w_spec, w_spec, b_spec, b_spec],
        out_specs=row_spec,
        out_shape=jax.ShapeDtypeStruct((_N, _D), jnp.float32),
    )(sumf, cntf, sumb, cntb, x,
      Wl, Wr, WlT, WrT, bl.reshape(1, _D), blT.reshape(1, _D))


@functools.lru_cache(maxsize=None)
def _make_score(ES: int):
    total_chunks = ES // _CHUNK
    per_tile = -(-total_chunks // _NW)
    mesh = plsc.VectorSubcoreMesh(core_axis_name="c", subcore_axis_name="s")

    scratch = [
        pltpu.VMEM((_CHUNK,), jnp.int32),
        pltpu.VMEM((_CHUNK,), jnp.int32),
        pltpu.VMEM((_CHUNK, _D), jnp.float32),
        pltpu.VMEM((_CHUNK, _D), jnp.float32),
        pltpu.VMEM((_CHUNK,), jnp.float32),
        pltpu.SemaphoreType.DMA,
        pltpu.SemaphoreType.DMA,
    ]

    def body(h_hbm, se_hbm, out_hbm, sidx, didx, hs, hd, ov, sem1, sem2):
        c = lax.axis_index("c")
        s = lax.axis_index("s")
        w = s * _NC + c
        lanes = lax.iota(jnp.int32, _L)

        def step(j, carry):
            chunk = w + _NW * j

            @pl.when(chunk < total_chunks)
            def _():
                off = chunk * _CHUNK
                pltpu.sync_copy(se_hbm.at[0, pl.ds(off, _CHUNK)], sidx)
                pltpu.sync_copy(se_hbm.at[1, pl.ds(off, _CHUNK)], didx)
                cp1 = pltpu.async_copy(h_hbm.at[sidx], hs, sem1)
                cp2 = pltpu.async_copy(h_hbm.at[didx], hd, sem2)
                cp1.wait()
                cp2.wait()
                for g in range(_CHUNK // _L):
                    vec = jnp.zeros((_L,), jnp.float32)
                    for l in range(_L):
                        e = g * _L + l
                        acc = hs[e, pl.ds(0, _L)] * hd[e, pl.ds(0, _L)]
                        for k in range(1, _D // _L):
                            acc = acc + (hs[e, pl.ds(k * _L, _L)]
                                         * hd[e, pl.ds(k * _L, _L)])
                        vec = jnp.where(lanes == l, jnp.sum(acc), vec)
                    ov[pl.ds(g * _L, _L)] = vec
                pltpu.sync_copy(ov, out_hbm.at[pl.ds(off, _CHUNK)])

            return carry

        lax.fori_loop(0, per_tile, step, 0)

    return pl.kernel(body, out_type=jax.ShapeDtypeStruct((ES,), jnp.float32),
                     mesh=mesh, scratch_types=tuple(scratch))


def kernel(node_embeddings, message_passing_edge_index, supervision_edge_index,
           Wl1, bl1, Wr1, Wl1T, bl1T, Wr1T, Wl2, bl2, Wr2, Wl2T, bl2T, Wr2T):
    x = node_embeddings
    ei = message_passing_edge_index
    E = ei.shape[1]
    ES = supervision_edge_index.shape[1]
    rows_per_tile = _N // _NS
    zrow = jnp.zeros((rows_per_tile, _D), jnp.float32)
    zcnt = jnp.zeros((rows_per_tile, _L), jnp.float32)

    sumf, sumb, cntf, cntb = _make_agg(E, True)(x, ei, zrow, zcnt)
    h = _layer_tc(sumf, cntf, sumb, cntb, x, Wl1, Wr1, Wl1T, Wr1T, bl1, bl1T)
    sumf2, sumb2 = _make_agg(E, False)(h, ei, zrow, zcnt)
    h2 = _layer_tc(sumf2, cntf, sumb2, cntb, h, Wl2, Wr2, Wl2T, Wr2T, bl2, bl2T)
    return _make_score(ES)(h2, supervision_edge_index)


# speedup vs baseline: 3.6568x; 3.6568x over previous
import functools

import jax
import jax.numpy as jnp
from jax import lax
from jax.experimental import pallas as pl
from jax.experimental.pallas import tpu as pltpu
from jax.experimental.pallas import tpu_sc as plsc

_N = 10000
_NP = 10240
_HALF = _NP // 2
_D = 128
_L = 16
_NC = 2
_NS = 16
_NW = _NC * _NS
_CHUNK = 128
_KSUB = 8
_SCH = 80


def _mesh():
    return plsc.VectorSubcoreMesh(core_axis_name="c", subcore_axis_name="s",
                                  num_cores=_NC, num_subcores=_NS)


@functools.lru_cache(maxsize=None)
def _make_agg_full(EROWS: int):
    n_blocks = EROWS // _KSUB
    per_tile = -(-n_blocks // _NS)
    rpt = _NP // _NS
    nacc = _NP + 8

    out_type = [
        jax.ShapeDtypeStruct((_NC, _NP, _D), jnp.float32),
        jax.ShapeDtypeStruct((_NC, _NP, _D), jnp.float32),
    ]
    scratch = [
        pltpu.VMEM((_KSUB, _CHUNK), jnp.int32),
        pltpu.VMEM((_KSUB, _CHUNK), jnp.int32),
        pltpu.VMEM((_CHUNK, _D), jnp.float32),
        pltpu.VMEM((_CHUNK, _D), jnp.float32),
        pltpu.VMEM_SHARED((nacc, _D), jnp.float32),
        pltpu.VMEM((80, _D), jnp.float32),
        pltpu.SemaphoreType.DMA,
    ]

    def body(x_hbm, gath_hbm, scat_hbm, ones_hbm, zrow_hbm,
             sums, cnts, gidx, sidx, rows, ones, acc_sh, stripe, sem):
        c = lax.axis_index("c")
        s = lax.axis_index("s")
        row0 = s * rpt
        pltpu.sync_copy(ones_hbm, ones)

        def zero_acc():
            pltpu.sync_copy(zrow_hbm, stripe)
            for k in range(rpt // 80):
                pltpu.sync_copy(stripe, acc_sh.at[pl.ds(row0 + k * 80, 80)])

        def write_out(out_ref):
            for k in range(rpt // 80):
                ws = pl.ds(row0 + k * 80, 80)
                pltpu.sync_copy(acc_sh.at[ws], stripe)
                pltpu.sync_copy(stripe, out_ref.at[c, ws])

        zero_acc()
        plsc.subcore_barrier()

        def cstep(j, carry):
            blk = j * _NS + s

            @pl.when(blk < n_blocks)
            def _():
                r0 = blk * _KSUB
                pltpu.sync_copy(scat_hbm.at[c, pl.ds(r0, _KSUB)], sidx)
                for k in range(_KSUB):
                    pltpu.sync_copy(ones, acc_sh.at[sidx.at[k]], add=True)

            return carry

        lax.fori_loop(0, per_tile, cstep, 0)
        plsc.subcore_barrier()
        write_out(cnts)

        zero_acc()
        plsc.subcore_barrier()

        def step(j, carry):
            blk = j * _NS + s

            @pl.when(blk < n_blocks)
            def _():
                r0 = blk * _KSUB
                pltpu.sync_copy(gath_hbm.at[c, pl.ds(r0, _KSUB)], gidx)
                pltpu.sync_copy(scat_hbm.at[c, pl.ds(r0, _KSUB)], sidx)
                for k in range(_KSUB):
                    pltpu.async_copy(x_hbm.at[gidx.at[k]], rows, sem).wait()
                    pltpu.sync_copy(rows, acc_sh.at[sidx.at[k]], add=True)

            return carry

        lax.fori_loop(0, per_tile, step, 0)
        plsc.subcore_barrier()
        write_out(sums)

    return pl.kernel(body, out_type=tuple(out_type), mesh=_mesh(),
                     scratch_types=tuple(scratch))


@functools.lru_cache(maxsize=None)
def _make_agg_half(EROWS: int):
    n_blocks = EROWS // _KSUB
    per_tile = -(-n_blocks // _NS)
    rpt = _HALF // _NS

    out_type = [
        jax.ShapeDtypeStruct((_NC, _NP, _D), jnp.float32),
    ]
    scratch = [
        pltpu.VMEM((_KSUB, _CHUNK), jnp.int32),
        pltpu.VMEM((_KSUB, _CHUNK), jnp.int32),
        pltpu.VMEM((_CHUNK, _D), jnp.float32),
        pltpu.VMEM_SHARED((_HALF + 8, _D), jnp.float32),
        pltpu.VMEM((80, _D), jnp.float32),
        pltpu.SemaphoreType.DMA,
    ]

    def body(x_hbm, gath_hbm, scat0_hbm, scat1_hbm, zrow_hbm,
             sums, gidx, sidx, rows, acc_sh, stripe, sem):
        c = lax.axis_index("c")
        s = lax.axis_index("s")
        row0 = s * rpt

        for phase in range(2):
            base = phase * _HALF
            scat_hbm = scat0_hbm if phase == 0 else scat1_hbm
            pltpu.sync_copy(zrow_hbm, stripe)
            for k in range(rpt // 80):
                pltpu.sync_copy(stripe, acc_sh.at[pl.ds(row0 + k * 80, 80)])
            plsc.subcore_barrier()

            def step(j, carry):
                blk = j * _NS + s

                @pl.when(blk < n_blocks)
                def _():
                    r0 = blk * _KSUB
                    pltpu.sync_copy(gath_hbm.at[c, pl.ds(r0, _KSUB)], gidx)
                    pltpu.sync_copy(scat_hbm.at[c, pl.ds(r0, _KSUB)], sidx)
                    for k in range(_KSUB):
                        pltpu.async_copy(x_hbm.at[gidx.at[k]], rows,
                                         sem).wait()
                        pltpu.sync_copy(rows, acc_sh.at[sidx.at[k]],
                                        add=True)

                return carry

            lax.fori_loop(0, per_tile, step, 0)
            plsc.subcore_barrier()

            for k in range(rpt // 80):
                ws = pl.ds(row0 + k * 80, 80)
                os = pl.ds(base + row0 + k * 80, 80)
                pltpu.sync_copy(acc_sh.at[ws], stripe)
                pltpu.sync_copy(stripe, sums.at[c, os])

    return pl.kernel(body, out_type=tuple(out_type), mesh=_mesh(),
                     scratch_types=tuple(scratch))


def _layer_tc(sumf, cntf, sumb, cntb, x, Wl, Wr, WlT, WrT, bl, blT):
    BR = 1000
    grid = (_N // BR,)
    dn = (((1,), (1,)), ((), ()))

    def body(sf, cf, sb, cb, xr, wl, wr, wlt, wrt, b1, b2, o):
        inv_f = 1.0 / jnp.maximum(cf[:, 0:1], 1.0)
        inv_b = 1.0 / jnp.maximum(cb[:, 0:1], 1.0)
        hf = lax.dot_general(sf[...] * inv_f, wl[...], dn,
                             preferred_element_type=jnp.float32)
        hb = lax.dot_general(sb[...] * inv_b, wlt[...], dn,
                             preferred_element_type=jnp.float32)
        hx = lax.dot_general(xr[...], wr[...] + wrt[...], dn,
                             preferred_element_type=jnp.float32)
        o[...] = hf + hb + hx + (b1[...] + b2[...])

    row_spec = pl.BlockSpec((BR, _D), lambda i: (i, 0))
    cnt_spec = pl.BlockSpec((BR, _D), lambda i: (i, 0))
    w_spec = pl.BlockSpec((_D, _D), lambda i: (0, 0))
    b_spec = pl.BlockSpec((1, _D), lambda i: (0, 0))
    return pl.pallas_call(
        body,
        grid=grid,
        in_specs=[row_spec, cnt_spec, row_spec, cnt_spec, row_spec,
                  w_spec, w_spec, w_spec, w_spec, b_spec, b_spec],
        out_specs=row_spec,
        out_shape=jax.ShapeDtypeStruct((_N, _D), jnp.float32),
    )(sumf, cntf, sumb, cntb, x,
      Wl, Wr, WlT, WrT, bl.reshape(1, _D), blT.reshape(1, _D))


def _lane_gather(v, idx):
    dn = lax.GatherDimensionNumbers(offset_dims=(), collapsed_slice_dims=(0,),
                                    start_index_map=(0,))
    return lax.gather(v, idx[:, None], dn, (1,),
                      mode=lax.GatherScatterMode.PROMISE_IN_BOUNDS)


@functools.lru_cache(maxsize=None)
def _make_score(ES: int):
    total_chunks = ES // _SCH
    per_tile = -(-total_chunks // _NW)

    scratch = [
        pltpu.VMEM((_SCH,), jnp.int32),
        pltpu.VMEM((_SCH,), jnp.int32),
        pltpu.VMEM((_SCH, _D), jnp.float32),
        pltpu.VMEM((_SCH, _D), jnp.float32),
        pltpu.VMEM((_SCH,), jnp.float32),
        pltpu.SemaphoreType.DMA,
        pltpu.SemaphoreType.DMA,
    ]

    def body(h_hbm, ssrc_hbm, sdst_hbm, out_hbm, sidx, didx, hs, hd, ov,
             sem1, sem2):
        c = lax.axis_index("c")
        s = lax.axis_index("s")
        w = s * _NC + c
        lanes = lax.iota(jnp.int32, _L)
        perms = [lanes ^ k for k in (1, 2, 4, 8)]

        def step(j, carry):
            chunk = w + _NW * j

            @pl.when(chunk < total_chunks)
            def _():
                off = chunk * _SCH
                pltpu.sync_copy(ssrc_hbm.at[pl.ds(off, _SCH)], sidx)
                pltpu.sync_copy(sdst_hbm.at[pl.ds(off, _SCH)], didx)
                cp1 = pltpu.async_copy(h_hbm.at[sidx], hs, sem1)
                cp2 = pltpu.async_copy(h_hbm.at[didx], hd, sem2)
                cp1.wait()
                cp2.wait()
                for g in range(_SCH // _L):
                    vec = jnp.zeros((_L,), jnp.float32)
                    for l in range(_L):
                        e = g * _L + l
                        acc = hs[e, pl.ds(0, _L)] * hd[e, pl.ds(0, _L)]
                        for k in range(1, _D // _L):
                            acc = acc + (hs[e, pl.ds(k * _L, _L)]
                                         * hd[e, pl.ds(k * _L, _L)])
                        for pm in perms:
                            acc = acc + _lane_gather(acc, pm)
                        vec = jnp.where(lanes == l, acc, vec)
                    ov[pl.ds(g * _L, _L)] = vec
                pltpu.sync_copy(ov, out_hbm.at[pl.ds(off, _SCH)])

            return carry

        lax.fori_loop(0, per_tile, step, 0)

    return pl.kernel(body, out_type=jax.ShapeDtypeStruct((ES,), jnp.float32),
                     mesh=_mesh(), scratch_types=tuple(scratch))


def kernel(node_embeddings, message_passing_edge_index, supervision_edge_index,
           Wl1, bl1, Wr1, Wl1T, bl1T, Wr1T, Wl2, bl2, Wr2, Wl2T, bl2T, Wr2T):
    x = node_embeddings
    src = message_passing_edge_index[0]
    dst = message_passing_edge_index[1]
    s_src = supervision_edge_index[0]
    s_dst = supervision_edge_index[1]
    E = src.shape[0]
    ES = s_src.shape[0]

    blk_e = _CHUNK * _KSUB
    pad = (-E) % blk_e
    erows = (E + pad) // _CHUNK
    zpad = jnp.zeros((pad,), jnp.int32)
    dpad = jnp.full((pad,), _NP, jnp.int32)
    srcg = jnp.concatenate([src, zpad]).reshape(erows, _CHUNK)
    dstg = jnp.concatenate([dst, zpad]).reshape(erows, _CHUNK)
    srcs = jnp.concatenate([src, dpad]).reshape(erows, _CHUNK)
    dsts = jnp.concatenate([dst, dpad]).reshape(erows, _CHUNK)
    gath = jnp.stack([srcg, dstg])
    scat = jnp.stack([dsts, srcs])
    scat0 = jnp.where(scat < _HALF, scat, _HALF)
    scat1 = jnp.where((scat >= _HALF) & (scat < 2 * _HALF),
                      scat - _HALF, _HALF)

    zrow = jnp.zeros((80, _D), jnp.float32)
    ones = jnp.ones((_CHUNK, _D), jnp.float32)

    sums, cnts = _make_agg_full(erows)(x, gath, scat, ones, zrow)
    cntf, cntb = cnts[0], cnts[1]
    h = _layer_tc(sums[0], cntf, sums[1], cntb, x,
                  Wl1, Wr1, Wl1T, Wr1T, bl1, bl1T)
    (tsums,) = _make_agg_half(erows)(h, gath, scat0, scat1, zrow)
    h2 = _layer_tc(tsums[0], cntf, tsums[1], cntb, h,
                   Wl2, Wr2, Wl2T, Wr2T, bl2, bl2T)
    return _make_score(ES)(h2, s_src, s_dst)
